# baseline (speedup 1.0000x reference)
import functools

import jax
import jax.numpy as jnp
from jax import lax
from jax.experimental import pallas as pl
from jax.experimental.pallas import tpu as pltpu
from jax.experimental.pallas import tpu_sc as plsc

_N = 10000
_E = 320000
_D = 128
_G = 128
_EPS = 1e-5

_NC = 2
_NS = 16
_NW = _NC * _NS
_C = 128
_NCH = 79
_EP = _NW * _NCH * _C
_NSH = 10008
_RPS = 624
_RTAIL = _N - _NS * _RPS
_CNT_CH = 1000
_R = 400


def _zero_acc(z2_hbm, acc_sh, s):
    zoff = pl.multiple_of(s * _RPS, 8)
    pltpu.sync_copy(z2_hbm.at[pl.ds(zoff, _RPS)],
                    acc_sh.at[pl.ds(zoff, _RPS)])

    @pl.when(s == 0)
    def _():
        pltpu.sync_copy(z2_hbm.at[pl.ds(_NS * _RPS, _RTAIL)],
                        acc_sh.at[pl.ds(_NS * _RPS, _RTAIL)])


def _acc_out_copy(acc_sh, acc_out, c, s):
    zoff = pl.multiple_of(s * _RPS, 8)
    ooff = pl.multiple_of(c * _N + s * _RPS, 8)
    pltpu.sync_copy(acc_sh.at[pl.ds(zoff, _RPS)],
                    acc_out.at[pl.ds(ooff, _RPS)])

    @pl.when(s == 0)
    def _():
        toff = pl.multiple_of(c * _N + _NS * _RPS, 8)
        pltpu.sync_copy(acc_sh.at[pl.ds(_NS * _RPS, _RTAIL)],
                        acc_out.at[pl.ds(toff, _RTAIL)])


def _make_agg_body(with_cnt):
    def body(x_hbm, eidx_hbm, z2_hbm, *refs):
        if with_cnt:
            (z1_hbm, acc_out, cnt_out, idx_a, idx_b, rows_a, rows_b, ones_v,
             cntb_v, acc_sh, cnt_sh, isem_a, isem_b, gsem_a, gsem_b) = refs
        else:
            (acc_out, idx_a, idx_b, rows_a, rows_b, acc_sh,
             isem_a, isem_b, gsem_a, gsem_b) = refs
        c = lax.axis_index("c")
        s = lax.axis_index("s")
        wid = c * _NS + s
        base = wid * _NCH

        pltpu.async_copy(eidx_hbm.at[base], idx_a, isem_a)
        pltpu.async_copy(eidx_hbm.at[base + 1], idx_b, isem_b)

        _zero_acc(z2_hbm, acc_sh, s)

        if with_cnt:
            @pl.when(s < _N // _CNT_CH)
            def _():
                coff = pl.multiple_of(s * _CNT_CH, 8)
                pltpu.sync_copy(z1_hbm.at[pl.ds(coff, _CNT_CH)], cntb_v)
                pltpu.sync_copy(cntb_v, cnt_sh.at[pl.ds(coff, _CNT_CH)])

            ones16 = jnp.full((16,), 1.0, jnp.float32)
            for j in range(_C // 16):
                ones_v[pl.ds(j * 16, 16)] = ones16

        pltpu.make_async_copy(eidx_hbm.at[base], idx_a, isem_a).wait()
        pltpu.async_copy(x_hbm.at[idx_a.at[0]], rows_a, gsem_a)
        plsc.subcore_barrier()

        def scatter(rows, idx):
            pltpu.sync_copy(rows, acc_sh.at[idx.at[1]], add=True)
            if with_cnt:
                pltpu.sync_copy(ones_v, cnt_sh.at[idx.at[1]], add=True)

        def pair(k, carry):
            i0 = 2 * k
            pltpu.make_async_copy(eidx_hbm.at[base + i0 + 1], idx_b,
                                  isem_b).wait()
            pltpu.async_copy(x_hbm.at[idx_b.at[0]], rows_b, gsem_b)
            pltpu.make_async_copy(x_hbm.at[idx_a.at[0]], rows_a,
                                  gsem_a).wait()
            scatter(rows_a, idx_a)
            pltpu.async_copy(eidx_hbm.at[base + i0 + 2], idx_a, isem_a)
            pltpu.make_async_copy(eidx_hbm.at[base + i0 + 2], idx_a,
                                  isem_a).wait()
            pltpu.async_copy(x_hbm.at[idx_a.at[0]], rows_a, gsem_a)
            pltpu.make_async_copy(x_hbm.at[idx_b.at[0]], rows_b,
                                  gsem_b).wait()
            scatter(rows_b, idx_b)

            @pl.when(k < _NCH // 2 - 1)
            def _():
                pltpu.async_copy(eidx_hbm.at[base + i0 + 3], idx_b, isem_b)

            return carry

        lax.fori_loop(0, _NCH // 2, pair, 0)

        pltpu.make_async_copy(x_hbm.at[idx_a.at[0]], rows_a, gsem_a).wait()
        scatter(rows_a, idx_a)

        plsc.subcore_barrier()

        _acc_out_copy(acc_sh, acc_out, c, s)

        if with_cnt:
            @pl.when(s < _N // _CNT_CH)
            def _():
                coff = pl.multiple_of(s * _CNT_CH, 8)
                off = pl.multiple_of(c * _N + s * _CNT_CH, 8)
                pltpu.sync_copy(cnt_sh.at[pl.ds(coff, _CNT_CH)], cntb_v)
                pltpu.sync_copy(cntb_v, cnt_out.at[pl.ds(off, _CNT_CH)])

    return body


@functools.lru_cache(maxsize=None)
def _build_agg(with_cnt):
    mesh = plsc.VectorSubcoreMesh(core_axis_name="c", subcore_axis_name="s",
                                  num_cores=_NC, num_subcores=_NS)
    if with_cnt:
        return pl.kernel(
            _make_agg_body(True),
            out_type=(jax.ShapeDtypeStruct((_NC * _N, _D), jnp.float32),
                      jax.ShapeDtypeStruct((_NC * _N,), jnp.float32)),
            mesh=mesh,
            scratch_types=[
                pltpu.VMEM((2, _C), jnp.int32),
                pltpu.VMEM((2, _C), jnp.int32),
                pltpu.VMEM((_C, _D), jnp.float32),
                pltpu.VMEM((_C, _D), jnp.float32),
                pltpu.VMEM((_C,), jnp.float32),
                pltpu.VMEM((_CNT_CH,), jnp.float32),
                pltpu.VMEM_SHARED((_NSH, _D), jnp.float32),
                pltpu.VMEM_SHARED((_NSH,), jnp.float32),
                pltpu.SemaphoreType.DMA,
                pltpu.SemaphoreType.DMA,
                pltpu.SemaphoreType.DMA,
                pltpu.SemaphoreType.DMA,
            ],
        )
    return pl.kernel(
        _make_agg_body(False),
        out_type=jax.ShapeDtypeStruct((_NC * _N, _D), jnp.float32),
        mesh=mesh,
        scratch_types=[
            pltpu.VMEM((2, _C), jnp.int32),
            pltpu.VMEM((2, _C), jnp.int32),
            pltpu.VMEM((_C, _D), jnp.float32),
            pltpu.VMEM((_C, _D), jnp.float32),
            pltpu.VMEM_SHARED((_NSH, _D), jnp.float32),
            pltpu.SemaphoreType.DMA,
            pltpu.SemaphoreType.DMA,
            pltpu.SemaphoreType.DMA,
            pltpu.SemaphoreType.DMA,
        ],
    )


def _pack_edges(edge_index):
    pad = _EP - _E
    srcp = jnp.concatenate([edge_index[0],
                            jnp.zeros((pad,), jnp.int32)])
    dstp = jnp.concatenate([edge_index[1],
                            jnp.full((pad,), _N, jnp.int32)])
    return jnp.stack([srcp.reshape(_NW * _NCH, _C),
                      dstp.reshape(_NW * _NCH, _C)], axis=1)


def _sc_aggregate(x, eidx, with_cnt):
    z2 = jnp.zeros((_N, _D), jnp.float32)
    if with_cnt:
        z1 = jnp.zeros((_N,), jnp.float32)
        acc, cnt = _build_agg(True)(x, eidx, z2, z1)
        return acc.reshape(_NC, _N, _D), cnt.reshape(_NC, _N, 1)
    acc = _build_agg(False)(x, eidx, z2)
    return acc.reshape(_NC, _N, _D)


def _combine_body(acc0_r, acc1_r, cnt0_r, cnt1_r, x_r, wl_r, bl_r, wr_r,
                  h_r, st_r):
    agg = acc0_r[0] + acc1_r[0]
    cnt = cnt0_r[0] + cnt1_r[0]
    mean = agg / jnp.maximum(cnt, 1.0)
    h = (jnp.dot(mean, wl_r[...], preferred_element_type=jnp.float32)
         + jnp.dot(x_r[...], wr_r[...], preferred_element_type=jnp.float32)
         + bl_r[...])
    h_r[...] = h
    ssum = jnp.sum(h, axis=0, keepdims=True)
    ssq = jnp.sum(h * h, axis=0, keepdims=True)
    st = jnp.concatenate([ssum, ssq, jnp.zeros((6, _D), jnp.float32)], axis=0)
    i = pl.program_id(0)

    @pl.when(i == 0)
    def _():
        st_r[...] = st

    @pl.when(i > 0)
    def _():
        st_r[...] = st_r[...] + st


def _tc_combine(acc, cnt, x, wl, bl, wr):
    return pl.pallas_call(
        _combine_body,
        grid=(_N // _R,),
        in_specs=[
            pl.BlockSpec((1, _R, _D), lambda i: (0, i, 0)),
            pl.BlockSpec((1, _R, _D), lambda i: (1, i, 0)),
            pl.BlockSpec((1, _R, 1), lambda i: (0, i, 0)),
            pl.BlockSpec((1, _R, 1), lambda i: (1, i, 0)),
            pl.BlockSpec((_R, _D), lambda i: (i, 0)),
            pl.BlockSpec((_D, _D), lambda i: (0, 0)),
            pl.BlockSpec((1, _D), lambda i: (0, 0)),
            pl.BlockSpec((_D, _D), lambda i: (0, 0)),
        ],
        out_specs=[
            pl.BlockSpec((_R, _D), lambda i: (i, 0)),
            pl.BlockSpec((8, _D), lambda i: (0, 0)),
        ],
        out_shape=[jax.ShapeDtypeStruct((_N, _D), jnp.float32),
                   jax.ShapeDtypeStruct((8, _D), jnp.float32)],
    )(acc, acc, cnt, cnt, x, wl, bl, wr)


def _norm_body(h_r, st_r, g_r, be_r, o_r):
    st = st_r[...]
    mu = st[0:1, :] * (1.0 / _N)
    var = st[1:2, :] * (1.0 / _N) - mu * mu
    inv = lax.rsqrt(var + _EPS)
    o_r[...] = jnp.maximum(g_r[...] * (h_r[...] - mu) * inv + be_r[...], 0.0)


def _tc_norm(h, st, g, be):
    return pl.pallas_call(
        _norm_body,
        grid=(_N // _R,),
        in_specs=[
            pl.BlockSpec((_R, _D), lambda i: (i, 0)),
            pl.BlockSpec((8, _D), lambda i: (0, 0)),
            pl.BlockSpec((1, _D), lambda i: (0, 0)),
            pl.BlockSpec((1, _D), lambda i: (0, 0)),
        ],
        out_specs=pl.BlockSpec((_R, _D), lambda i: (i, 0)),
        out_shape=jax.ShapeDtypeStruct((_N, _D), jnp.float32),
    )(h, st, g, be)


def _final_body(h_r, st_r, g_r, be_r, b_r, w1_r, b1_r, w2_r, b2_r,
                o_r, pool_sc):
    i = pl.program_id(0)
    st = st_r[...]
    mu = st[0:1, :] * (1.0 / _N)
    var = st[1:2, :] * (1.0 / _N) - mu * mu
    h2 = jnp.maximum(
        g_r[...] * (h_r[...] - mu) * lax.rsqrt(var + _EPS) + be_r[...], 0.0)
    onehot = (b_r[...] == lax.broadcasted_iota(jnp.int32, (_R, _G), 1)
              ).astype(jnp.float32)
    p = lax.dot_general(onehot, h2, (((0,), (0,)), ((), ())),
                        preferred_element_type=jnp.float32)

    @pl.when(i == 0)
    def _():
        pool_sc[...] = p

    @pl.when(i > 0)
    def _():
        pool_sc[...] = pool_sc[...] + p

    @pl.when(i == pl.num_programs(0) - 1)
    def _():
        t = jnp.maximum(
            jnp.dot(pool_sc[...], w1_r[...],
                    preferred_element_type=jnp.float32) + b1_r[...], 0.0)
        o_r[...] = (jnp.dot(t, w2_r[...], preferred_element_type=jnp.float32)
                    + b2_r[...])


def _tc_final(h, st, g, be, batch2d, w1, b1, w2, b2):
    return pl.pallas_call(
        _final_body,
        grid=(_N // _R,),
        in_specs=[
            pl.BlockSpec((_R, _D), lambda i: (i, 0)),
            pl.BlockSpec((8, _D), lambda i: (0, 0)),
            pl.BlockSpec((1, _D), lambda i: (0, 0)),
            pl.BlockSpec((1, _D), lambda i: (0, 0)),
            pl.BlockSpec((_R, 1), lambda i: (i, 0)),
            pl.BlockSpec((_D, _D // 2), lambda i: (0, 0)),
            pl.BlockSpec((1, _D // 2), lambda i: (0, 0)),
            pl.BlockSpec((_D // 2, 1), lambda i: (0, 0)),
            pl.BlockSpec((1, 1), lambda i: (0, 0)),
        ],
        out_specs=pl.BlockSpec((_G, 1), lambda i: (0, 0)),
        out_shape=jax.ShapeDtypeStruct((_G, 1), jnp.float32),
        scratch_shapes=[pltpu.VMEM((_G, _G), jnp.float32)],
    )(h, st, g, be, batch2d, w1, b1, w2, b2)


def kernel(x, edge_index, batch, Wl0, bl0, Wr0, g0, be0,
           Wl1, bl1, Wr1, g1, be1, fcW1, fcb1, fcW2, fcb2):
    eidx = _pack_edges(edge_index)

    acc0, cnt = _sc_aggregate(x, eidx, True)
    h0, st0 = _tc_combine(acc0, cnt, x, Wl0, bl0.reshape(1, _D), Wr0)
    h1 = _tc_norm(h0, st0, g0.reshape(1, _D), be0.reshape(1, _D))

    acc1 = _sc_aggregate(h1, eidx, False)
    h2, st1 = _tc_combine(acc1, cnt, h1, Wl1, bl1.reshape(1, _D), Wr1)

    return _tc_final(h2, st1, g1.reshape(1, _D), be1.reshape(1, _D),
                     batch.reshape(_N, 1), fcW1, fcb1.reshape(1, _D // 2),
                     fcW2, fcb2.reshape(1, 1))


# speedup vs baseline: 1.0004x; 1.0004x over previous
import functools

import jax
import jax.numpy as jnp
from jax import lax
from jax.experimental import pallas as pl
from jax.experimental.pallas import tpu as pltpu
from jax.experimental.pallas import tpu_sc as plsc

_N = 10000
_E = 320000
_D = 128
_G = 128
_EPS = 1e-5

_NC = 2
_NS = 16
_NW = _NC * _NS
_C = 128
_NCH = 79
_EP = _NW * _NCH * _C
_NSH = 10512
_RPS = 624
_RTAIL = _N - _NS * _RPS
_CNT_CH = 1000
_R = 400


def _zero_acc(z2_hbm, acc_sh, s):
    zoff = pl.multiple_of(s * _RPS, 8)
    pltpu.sync_copy(z2_hbm.at[pl.ds(zoff, _RPS)],
                    acc_sh.at[pl.ds(zoff, _RPS)])

    @pl.when(s == 0)
    def _():
        pltpu.sync_copy(z2_hbm.at[pl.ds(_NS * _RPS, _RTAIL)],
                        acc_sh.at[pl.ds(_NS * _RPS, _RTAIL)])


def _acc_out_copy(acc_sh, acc_out, c, s):
    zoff = pl.multiple_of(s * _RPS, 8)
    ooff = pl.multiple_of(c * _N + s * _RPS, 8)
    pltpu.sync_copy(acc_sh.at[pl.ds(zoff, _RPS)],
                    acc_out.at[pl.ds(ooff, _RPS)])

    @pl.when(s == 0)
    def _():
        toff = pl.multiple_of(c * _N + _NS * _RPS, 8)
        pltpu.sync_copy(acc_sh.at[pl.ds(_NS * _RPS, _RTAIL)],
                        acc_out.at[pl.ds(toff, _RTAIL)])


def _make_agg_body(with_cnt):
    def body(x_hbm, eidx_hbm, z2_hbm, *refs):
        if with_cnt:
            (z1_hbm, acc_out, cnt_out, idx_a, idx_b, rows_a, rows_b, ones_v,
             cntb_v, acc_sh, cnt_sh, isem_a, isem_b, gsem_a, gsem_b) = refs
        else:
            (acc_out, idx_a, idx_b, rows_a, rows_b, acc_sh,
             isem_a, isem_b, gsem_a, gsem_b) = refs
        c = lax.axis_index("c")
        s = lax.axis_index("s")
        wid = c * _NS + s
        base = wid * _NCH

        pltpu.async_copy(eidx_hbm.at[base], idx_a, isem_a)
        pltpu.async_copy(eidx_hbm.at[base + 1], idx_b, isem_b)

        _zero_acc(z2_hbm, acc_sh, s)

        if with_cnt:
            @pl.when(s < _N // _CNT_CH)
            def _():
                coff = pl.multiple_of(s * _CNT_CH, 8)
                pltpu.sync_copy(z1_hbm.at[pl.ds(coff, _CNT_CH)], cntb_v)
                pltpu.sync_copy(cntb_v, cnt_sh.at[pl.ds(coff, _CNT_CH)])

            ones16 = jnp.full((16,), 1.0, jnp.float32)
            for j in range(_C // 16):
                ones_v[pl.ds(j * 16, 16)] = ones16

        pltpu.make_async_copy(eidx_hbm.at[base], idx_a, isem_a).wait()
        pltpu.async_copy(x_hbm.at[idx_a.at[0]], rows_a, gsem_a)
        plsc.subcore_barrier()

        def scatter(rows, idx):
            pltpu.sync_copy(rows, acc_sh.at[idx.at[1]], add=True)
            if with_cnt:
                pltpu.sync_copy(ones_v, cnt_sh.at[idx.at[1]], add=True)

        def pair(k, carry):
            i0 = 2 * k
            pltpu.make_async_copy(eidx_hbm.at[base + i0 + 1], idx_b,
                                  isem_b).wait()
            pltpu.async_copy(x_hbm.at[idx_b.at[0]], rows_b, gsem_b)
            pltpu.make_async_copy(x_hbm.at[idx_a.at[0]], rows_a,
                                  gsem_a).wait()
            scatter(rows_a, idx_a)
            pltpu.async_copy(eidx_hbm.at[base + i0 + 2], idx_a, isem_a)
            pltpu.make_async_copy(eidx_hbm.at[base + i0 + 2], idx_a,
                                  isem_a).wait()
            pltpu.async_copy(x_hbm.at[idx_a.at[0]], rows_a, gsem_a)
            pltpu.make_async_copy(x_hbm.at[idx_b.at[0]], rows_b,
                                  gsem_b).wait()
            scatter(rows_b, idx_b)

            @pl.when(k < _NCH // 2 - 1)
            def _():
                pltpu.async_copy(eidx_hbm.at[base + i0 + 3], idx_b, isem_b)

            return carry

        lax.fori_loop(0, _NCH // 2, pair, 0)

        pltpu.make_async_copy(x_hbm.at[idx_a.at[0]], rows_a, gsem_a).wait()
        scatter(rows_a, idx_a)

        plsc.subcore_barrier()

        _acc_out_copy(acc_sh, acc_out, c, s)

        if with_cnt:
            @pl.when(s < _N // _CNT_CH)
            def _():
                coff = pl.multiple_of(s * _CNT_CH, 8)
                off = pl.multiple_of(c * _N + s * _CNT_CH, 8)
                pltpu.sync_copy(cnt_sh.at[pl.ds(coff, _CNT_CH)], cntb_v)
                pltpu.sync_copy(cntb_v, cnt_out.at[pl.ds(off, _CNT_CH)])

    return body


@functools.lru_cache(maxsize=None)
def _build_agg(with_cnt):
    mesh = plsc.VectorSubcoreMesh(core_axis_name="c", subcore_axis_name="s",
                                  num_cores=_NC, num_subcores=_NS)
    if with_cnt:
        return pl.kernel(
            _make_agg_body(True),
            out_type=(jax.ShapeDtypeStruct((_NC * _N, _D), jnp.float32),
                      jax.ShapeDtypeStruct((_NC * _N,), jnp.float32)),
            mesh=mesh,
            scratch_types=[
                pltpu.VMEM((2, _C), jnp.int32),
                pltpu.VMEM((2, _C), jnp.int32),
                pltpu.VMEM((_C, _D), jnp.float32),
                pltpu.VMEM((_C, _D), jnp.float32),
                pltpu.VMEM((_C,), jnp.float32),
                pltpu.VMEM((_CNT_CH,), jnp.float32),
                pltpu.VMEM_SHARED((_NSH, _D), jnp.float32),
                pltpu.VMEM_SHARED((_NSH,), jnp.float32),
                pltpu.SemaphoreType.DMA,
                pltpu.SemaphoreType.DMA,
                pltpu.SemaphoreType.DMA,
                pltpu.SemaphoreType.DMA,
            ],
        )
    return pl.kernel(
        _make_agg_body(False),
        out_type=jax.ShapeDtypeStruct((_NC * _N, _D), jnp.float32),
        mesh=mesh,
        scratch_types=[
            pltpu.VMEM((2, _C), jnp.int32),
            pltpu.VMEM((2, _C), jnp.int32),
            pltpu.VMEM((_C, _D), jnp.float32),
            pltpu.VMEM((_C, _D), jnp.float32),
            pltpu.VMEM_SHARED((_NSH, _D), jnp.float32),
            pltpu.SemaphoreType.DMA,
            pltpu.SemaphoreType.DMA,
            pltpu.SemaphoreType.DMA,
            pltpu.SemaphoreType.DMA,
        ],
    )


def _pack_edges(edge_index):
    pad = _EP - _E
    srcp = jnp.concatenate([edge_index[0],
                            jnp.zeros((pad,), jnp.int32)])
    dstp = jnp.concatenate([edge_index[1],
                            _N + jnp.arange(pad, dtype=jnp.int32) % (_NSH - _N)])
    return jnp.stack([srcp.reshape(_NW * _NCH, _C),
                      dstp.reshape(_NW * _NCH, _C)], axis=1)


def _sc_aggregate(x, eidx, with_cnt):
    z2 = jnp.zeros((_N, _D), jnp.float32)
    if with_cnt:
        z1 = jnp.zeros((_N,), jnp.float32)
        acc, cnt = _build_agg(True)(x, eidx, z2, z1)
        return acc.reshape(_NC, _N, _D), cnt.reshape(_NC, _N, 1)
    acc = _build_agg(False)(x, eidx, z2)
    return acc.reshape(_NC, _N, _D)


def _combine_body(acc0_r, acc1_r, cnt0_r, cnt1_r, x_r, wl_r, bl_r, wr_r,
                  h_r, st_r):
    agg = acc0_r[0] + acc1_r[0]
    cnt = cnt0_r[0] + cnt1_r[0]
    mean = agg / jnp.maximum(cnt, 1.0)
    h = (jnp.dot(mean, wl_r[...], preferred_element_type=jnp.float32)
         + jnp.dot(x_r[...], wr_r[...], preferred_element_type=jnp.float32)
         + bl_r[...])
    h_r[...] = h
    ssum = jnp.sum(h, axis=0, keepdims=True)
    ssq = jnp.sum(h * h, axis=0, keepdims=True)
    st = jnp.concatenate([ssum, ssq, jnp.zeros((6, _D), jnp.float32)], axis=0)
    i = pl.program_id(0)

    @pl.when(i == 0)
    def _():
        st_r[...] = st

    @pl.when(i > 0)
    def _():
        st_r[...] = st_r[...] + st


def _tc_combine(acc, cnt, x, wl, bl, wr):
    return pl.pallas_call(
        _combine_body,
        grid=(_N // _R,),
        in_specs=[
            pl.BlockSpec((1, _R, _D), lambda i: (0, i, 0)),
            pl.BlockSpec((1, _R, _D), lambda i: (1, i, 0)),
            pl.BlockSpec((1, _R, 1), lambda i: (0, i, 0)),
            pl.BlockSpec((1, _R, 1), lambda i: (1, i, 0)),
            pl.BlockSpec((_R, _D), lambda i: (i, 0)),
            pl.BlockSpec((_D, _D), lambda i: (0, 0)),
            pl.BlockSpec((1, _D), lambda i: (0, 0)),
            pl.BlockSpec((_D, _D), lambda i: (0, 0)),
        ],
        out_specs=[
            pl.BlockSpec((_R, _D), lambda i: (i, 0)),
            pl.BlockSpec((8, _D), lambda i: (0, 0)),
        ],
        out_shape=[jax.ShapeDtypeStruct((_N, _D), jnp.float32),
                   jax.ShapeDtypeStruct((8, _D), jnp.float32)],
    )(acc, acc, cnt, cnt, x, wl, bl, wr)


def _norm_body(h_r, st_r, g_r, be_r, o_r):
    st = st_r[...]
    mu = st[0:1, :] * (1.0 / _N)
    var = st[1:2, :] * (1.0 / _N) - mu * mu
    inv = lax.rsqrt(var + _EPS)
    o_r[...] = jnp.maximum(g_r[...] * (h_r[...] - mu) * inv + be_r[...], 0.0)


def _tc_norm(h, st, g, be):
    return pl.pallas_call(
        _norm_body,
        grid=(_N // _R,),
        in_specs=[
            pl.BlockSpec((_R, _D), lambda i: (i, 0)),
            pl.BlockSpec((8, _D), lambda i: (0, 0)),
            pl.BlockSpec((1, _D), lambda i: (0, 0)),
            pl.BlockSpec((1, _D), lambda i: (0, 0)),
        ],
        out_specs=pl.BlockSpec((_R, _D), lambda i: (i, 0)),
        out_shape=jax.ShapeDtypeStruct((_N, _D), jnp.float32),
    )(h, st, g, be)


def _final_body(h_r, st_r, g_r, be_r, b_r, w1_r, b1_r, w2_r, b2_r,
                o_r, pool_sc):
    i = pl.program_id(0)
    st = st_r[...]
    mu = st[0:1, :] * (1.0 / _N)
    var = st[1:2, :] * (1.0 / _N) - mu * mu
    h2 = jnp.maximum(
        g_r[...] * (h_r[...] - mu) * lax.rsqrt(var + _EPS) + be_r[...], 0.0)
    onehot = (b_r[...] == lax.broadcasted_iota(jnp.int32, (_R, _G), 1)
              ).astype(jnp.float32)
    p = lax.dot_general(onehot, h2, (((0,), (0,)), ((), ())),
                        preferred_element_type=jnp.float32)

    @pl.when(i == 0)
    def _():
        pool_sc[...] = p

    @pl.when(i > 0)
    def _():
        pool_sc[...] = pool_sc[...] + p

    @pl.when(i == pl.num_programs(0) - 1)
    def _():
        t = jnp.maximum(
            jnp.dot(pool_sc[...], w1_r[...],
                    preferred_element_type=jnp.float32) + b1_r[...], 0.0)
        o_r[...] = (jnp.dot(t, w2_r[...], preferred_element_type=jnp.float32)
                    + b2_r[...])


def _tc_final(h, st, g, be, batch2d, w1, b1, w2, b2):
    return pl.pallas_call(
        _final_body,
        grid=(_N // _R,),
        in_specs=[
            pl.BlockSpec((_R, _D), lambda i: (i, 0)),
            pl.BlockSpec((8, _D), lambda i: (0, 0)),
            pl.BlockSpec((1, _D), lambda i: (0, 0)),
            pl.BlockSpec((1, _D), lambda i: (0, 0)),
            pl.BlockSpec((_R, 1), lambda i: (i, 0)),
            pl.BlockSpec((_D, _D // 2), lambda i: (0, 0)),
            pl.BlockSpec((1, _D // 2), lambda i: (0, 0)),
            pl.BlockSpec((_D // 2, 1), lambda i: (0, 0)),
            pl.BlockSpec((1, 1), lambda i: (0, 0)),
        ],
        out_specs=pl.BlockSpec((_G, 1), lambda i: (0, 0)),
        out_shape=jax.ShapeDtypeStruct((_G, 1), jnp.float32),
        scratch_shapes=[pltpu.VMEM((_G, _G), jnp.float32)],
    )(h, st, g, be, batch2d, w1, b1, w2, b2)


def kernel(x, edge_index, batch, Wl0, bl0, Wr0, g0, be0,
           Wl1, bl1, Wr1, g1, be1, fcW1, fcb1, fcW2, fcb2):
    eidx = _pack_edges(edge_index)

    acc0, cnt = _sc_aggregate(x, eidx, True)
    h0, st0 = _tc_combine(acc0, cnt, x, Wl0, bl0.reshape(1, _D), Wr0)
    h1 = _tc_norm(h0, st0, g0.reshape(1, _D), be0.reshape(1, _D))

    acc1 = _sc_aggregate(h1, eidx, False)
    h2, st1 = _tc_combine(acc1, cnt, h1, Wl1, bl1.reshape(1, _D), Wr1)

    return _tc_final(h2, st1, g1.reshape(1, _D), be1.reshape(1, _D),
                     batch.reshape(_N, 1), fcW1, fcb1.reshape(1, _D // 2),
                     fcW2, fcb2.reshape(1, 1))


# speedup vs baseline: 1.7000x; 1.6992x over previous
import functools

import jax
import jax.numpy as jnp
from jax import lax
from jax.experimental import pallas as pl
from jax.experimental.pallas import tpu as pltpu
from jax.experimental.pallas import tpu_sc as plsc

_N = 10000
_E = 320000
_D = 128
_G = 128
_EPS = 1e-5

_NC = 2
_NS = 16
_NW = _NC * _NS
_C = 80
_NCH = 125
_EP = _NW * _NCH * _C
_NSH = _N
_RPS = 624
_RTAIL = _N - _NS * _RPS
_CNT_CH = 1000
_R = 400


def _zero_acc(z2_hbm, acc_sh, s):
    zoff = pl.multiple_of(s * _RPS, 8)
    pltpu.sync_copy(z2_hbm.at[pl.ds(zoff, _RPS)],
                    acc_sh.at[pl.ds(zoff, _RPS)])

    @pl.when(s == 0)
    def _():
        pltpu.sync_copy(z2_hbm.at[pl.ds(_NS * _RPS, _RTAIL)],
                        acc_sh.at[pl.ds(_NS * _RPS, _RTAIL)])


def _acc_out_copy(acc_sh, acc_out, c, s):
    zoff = pl.multiple_of(s * _RPS, 8)
    ooff = pl.multiple_of(c * _N + s * _RPS, 8)
    pltpu.sync_copy(acc_sh.at[pl.ds(zoff, _RPS)],
                    acc_out.at[pl.ds(ooff, _RPS)])

    @pl.when(s == 0)
    def _():
        toff = pl.multiple_of(c * _N + _NS * _RPS, 8)
        pltpu.sync_copy(acc_sh.at[pl.ds(_NS * _RPS, _RTAIL)],
                        acc_out.at[pl.ds(toff, _RTAIL)])


def _make_agg_body(with_cnt):
    def body(x_hbm, eidx_hbm, z2_hbm, *refs):
        if with_cnt:
            (z1_hbm, acc_out, cnt_out, idx_a, idx_b, rows_a, rows_b, ones_v,
             cntb_v, acc_sh, cnt_sh, isem_a, isem_b, gsem_a, gsem_b) = refs
        else:
            (acc_out, idx_a, idx_b, rows_a, rows_b, acc_sh,
             isem_a, isem_b, gsem_a, gsem_b) = refs
        c = lax.axis_index("c")
        s = lax.axis_index("s")
        wid = c * _NS + s
        base = wid * _NCH

        pltpu.async_copy(eidx_hbm.at[base], idx_a, isem_a)
        pltpu.async_copy(eidx_hbm.at[base + 1], idx_b, isem_b)

        _zero_acc(z2_hbm, acc_sh, s)

        if with_cnt:
            @pl.when(s < _N // _CNT_CH)
            def _():
                coff = pl.multiple_of(s * _CNT_CH, 8)
                pltpu.sync_copy(z1_hbm.at[pl.ds(coff, _CNT_CH)], cntb_v)
                pltpu.sync_copy(cntb_v, cnt_sh.at[pl.ds(coff, _CNT_CH)])

            ones16 = jnp.full((16,), 1.0, jnp.float32)
            for j in range(_C // 16):
                ones_v[pl.ds(j * 16, 16)] = ones16

        pltpu.make_async_copy(eidx_hbm.at[base], idx_a, isem_a).wait()
        pltpu.async_copy(x_hbm.at[idx_a.at[0]], rows_a, gsem_a)
        plsc.subcore_barrier()

        def scatter(rows, idx):
            pltpu.sync_copy(rows, acc_sh.at[idx.at[1]], add=True)
            if with_cnt:
                pltpu.sync_copy(ones_v, cnt_sh.at[idx.at[1]], add=True)

        def pair(k, carry):
            i0 = 2 * k
            pltpu.make_async_copy(eidx_hbm.at[base + i0 + 1], idx_b,
                                  isem_b).wait()
            pltpu.async_copy(x_hbm.at[idx_b.at[0]], rows_b, gsem_b)
            pltpu.make_async_copy(x_hbm.at[idx_a.at[0]], rows_a,
                                  gsem_a).wait()
            scatter(rows_a, idx_a)
            pltpu.async_copy(eidx_hbm.at[base + i0 + 2], idx_a, isem_a)
            pltpu.make_async_copy(eidx_hbm.at[base + i0 + 2], idx_a,
                                  isem_a).wait()
            pltpu.async_copy(x_hbm.at[idx_a.at[0]], rows_a, gsem_a)
            pltpu.make_async_copy(x_hbm.at[idx_b.at[0]], rows_b,
                                  gsem_b).wait()
            scatter(rows_b, idx_b)

            @pl.when(k < _NCH // 2 - 1)
            def _():
                pltpu.async_copy(eidx_hbm.at[base + i0 + 3], idx_b, isem_b)

            return carry

        lax.fori_loop(0, _NCH // 2, pair, 0)

        pltpu.make_async_copy(x_hbm.at[idx_a.at[0]], rows_a, gsem_a).wait()
        scatter(rows_a, idx_a)

        plsc.subcore_barrier()

        _acc_out_copy(acc_sh, acc_out, c, s)

        if with_cnt:
            @pl.when(s < _N // _CNT_CH)
            def _():
                coff = pl.multiple_of(s * _CNT_CH, 8)
                off = pl.multiple_of(c * _N + s * _CNT_CH, 8)
                pltpu.sync_copy(cnt_sh.at[pl.ds(coff, _CNT_CH)], cntb_v)
                pltpu.sync_copy(cntb_v, cnt_out.at[pl.ds(off, _CNT_CH)])

    return body


@functools.lru_cache(maxsize=None)
def _build_agg(with_cnt):
    mesh = plsc.VectorSubcoreMesh(core_axis_name="c", subcore_axis_name="s",
                                  num_cores=_NC, num_subcores=_NS)
    if with_cnt:
        return pl.kernel(
            _make_agg_body(True),
            out_type=(jax.ShapeDtypeStruct((_NC * _N, _D), jnp.float32),
                      jax.ShapeDtypeStruct((_NC * _N,), jnp.float32)),
            mesh=mesh,
            scratch_types=[
                pltpu.VMEM((2, _C), jnp.int32),
                pltpu.VMEM((2, _C), jnp.int32),
                pltpu.VMEM((_C, _D), jnp.float32),
                pltpu.VMEM((_C, _D), jnp.float32),
                pltpu.VMEM((_C,), jnp.float32),
                pltpu.VMEM((_CNT_CH,), jnp.float32),
                pltpu.VMEM_SHARED((_NSH, _D), jnp.float32),
                pltpu.VMEM_SHARED((_NSH,), jnp.float32),
                pltpu.SemaphoreType.DMA,
                pltpu.SemaphoreType.DMA,
                pltpu.SemaphoreType.DMA,
                pltpu.SemaphoreType.DMA,
            ],
        )
    return pl.kernel(
        _make_agg_body(False),
        out_type=jax.ShapeDtypeStruct((_NC * _N, _D), jnp.float32),
        mesh=mesh,
        scratch_types=[
            pltpu.VMEM((2, _C), jnp.int32),
            pltpu.VMEM((2, _C), jnp.int32),
            pltpu.VMEM((_C, _D), jnp.float32),
            pltpu.VMEM((_C, _D), jnp.float32),
            pltpu.VMEM_SHARED((_NSH, _D), jnp.float32),
            pltpu.SemaphoreType.DMA,
            pltpu.SemaphoreType.DMA,
            pltpu.SemaphoreType.DMA,
            pltpu.SemaphoreType.DMA,
        ],
    )


def _pack_edges(edge_index):
    return jnp.stack([edge_index[0].reshape(_NW * _NCH, _C),
                      edge_index[1].reshape(_NW * _NCH, _C)], axis=1)


def _sc_aggregate(x, eidx, with_cnt):
    z2 = jnp.zeros((_N, _D), jnp.float32)
    if with_cnt:
        z1 = jnp.zeros((_N,), jnp.float32)
        acc, cnt = _build_agg(True)(x, eidx, z2, z1)
        return acc.reshape(_NC, _N, _D), cnt.reshape(_NC, _N, 1)
    acc = _build_agg(False)(x, eidx, z2)
    return acc.reshape(_NC, _N, _D)


def _combine_body(acc0_r, acc1_r, cnt0_r, cnt1_r, x_r, wl_r, bl_r, wr_r,
                  h_r, st_r):
    agg = acc0_r[0] + acc1_r[0]
    cnt = cnt0_r[0] + cnt1_r[0]
    mean = agg / jnp.maximum(cnt, 1.0)
    h = (jnp.dot(mean, wl_r[...], preferred_element_type=jnp.float32)
         + jnp.dot(x_r[...], wr_r[...], preferred_element_type=jnp.float32)
         + bl_r[...])
    h_r[...] = h
    ssum = jnp.sum(h, axis=0, keepdims=True)
    ssq = jnp.sum(h * h, axis=0, keepdims=True)
    st = jnp.concatenate([ssum, ssq, jnp.zeros((6, _D), jnp.float32)], axis=0)
    i = pl.program_id(0)

    @pl.when(i == 0)
    def _():
        st_r[...] = st

    @pl.when(i > 0)
    def _():
        st_r[...] = st_r[...] + st


def _tc_combine(acc, cnt, x, wl, bl, wr):
    return pl.pallas_call(
        _combine_body,
        grid=(_N // _R,),
        in_specs=[
            pl.BlockSpec((1, _R, _D), lambda i: (0, i, 0)),
            pl.BlockSpec((1, _R, _D), lambda i: (1, i, 0)),
            pl.BlockSpec((1, _R, 1), lambda i: (0, i, 0)),
            pl.BlockSpec((1, _R, 1), lambda i: (1, i, 0)),
            pl.BlockSpec((_R, _D), lambda i: (i, 0)),
            pl.BlockSpec((_D, _D), lambda i: (0, 0)),
            pl.BlockSpec((1, _D), lambda i: (0, 0)),
            pl.BlockSpec((_D, _D), lambda i: (0, 0)),
        ],
        out_specs=[
            pl.BlockSpec((_R, _D), lambda i: (i, 0)),
            pl.BlockSpec((8, _D), lambda i: (0, 0)),
        ],
        out_shape=[jax.ShapeDtypeStruct((_N, _D), jnp.float32),
                   jax.ShapeDtypeStruct((8, _D), jnp.float32)],
    )(acc, acc, cnt, cnt, x, wl, bl, wr)


def _norm_body(h_r, st_r, g_r, be_r, o_r):
    st = st_r[...]
    mu = st[0:1, :] * (1.0 / _N)
    var = st[1:2, :] * (1.0 / _N) - mu * mu
    inv = lax.rsqrt(var + _EPS)
    o_r[...] = jnp.maximum(g_r[...] * (h_r[...] - mu) * inv + be_r[...], 0.0)


def _tc_norm(h, st, g, be):
    return pl.pallas_call(
        _norm_body,
        grid=(_N // _R,),
        in_specs=[
            pl.BlockSpec((_R, _D), lambda i: (i, 0)),
            pl.BlockSpec((8, _D), lambda i: (0, 0)),
            pl.BlockSpec((1, _D), lambda i: (0, 0)),
            pl.BlockSpec((1, _D), lambda i: (0, 0)),
        ],
        out_specs=pl.BlockSpec((_R, _D), lambda i: (i, 0)),
        out_shape=jax.ShapeDtypeStruct((_N, _D), jnp.float32),
    )(h, st, g, be)


def _final_body(h_r, st_r, g_r, be_r, b_r, w1_r, b1_r, w2_r, b2_r,
                o_r, pool_sc):
    i = pl.program_id(0)
    st = st_r[...]
    mu = st[0:1, :] * (1.0 / _N)
    var = st[1:2, :] * (1.0 / _N) - mu * mu
    h2 = jnp.maximum(
        g_r[...] * (h_r[...] - mu) * lax.rsqrt(var + _EPS) + be_r[...], 0.0)
    onehot = (b_r[...] == lax.broadcasted_iota(jnp.int32, (_R, _G), 1)
              ).astype(jnp.float32)
    p = lax.dot_general(onehot, h2, (((0,), (0,)), ((), ())),
                        preferred_element_type=jnp.float32)

    @pl.when(i == 0)
    def _():
        pool_sc[...] = p

    @pl.when(i > 0)
    def _():
        pool_sc[...] = pool_sc[...] + p

    @pl.when(i == pl.num_programs(0) - 1)
    def _():
        t = jnp.maximum(
            jnp.dot(pool_sc[...], w1_r[...],
                    preferred_element_type=jnp.float32) + b1_r[...], 0.0)
        o_r[...] = (jnp.dot(t, w2_r[...], preferred_element_type=jnp.float32)
                    + b2_r[...])


def _tc_final(h, st, g, be, batch2d, w1, b1, w2, b2):
    return pl.pallas_call(
        _final_body,
        grid=(_N // _R,),
        in_specs=[
            pl.BlockSpec((_R, _D), lambda i: (i, 0)),
            pl.BlockSpec((8, _D), lambda i: (0, 0)),
            pl.BlockSpec((1, _D), lambda i: (0, 0)),
            pl.BlockSpec((1, _D), lambda i: (0, 0)),
            pl.BlockSpec((_R, 1), lambda i: (i, 0)),
            pl.BlockSpec((_D, _D // 2), lambda i: (0, 0)),
            pl.BlockSpec((1, _D // 2), lambda i: (0, 0)),
            pl.BlockSpec((_D // 2, 1), lambda i: (0, 0)),
            pl.BlockSpec((1, 1), lambda i: (0, 0)),
        ],
        out_specs=pl.BlockSpec((_G, 1), lambda i: (0, 0)),
        out_shape=jax.ShapeDtypeStruct((_G, 1), jnp.float32),
        scratch_shapes=[pltpu.VMEM((_G, _G), jnp.float32)],
    )(h, st, g, be, batch2d, w1, b1, w2, b2)


def kernel(x, edge_index, batch, Wl0, bl0, Wr0, g0, be0,
           Wl1, bl1, Wr1, g1, be1, fcW1, fcb1, fcW2, fcb2):
    eidx = _pack_edges(edge_index)

    acc0, cnt = _sc_aggregate(x, eidx, True)
    h0, st0 = _tc_combine(acc0, cnt, x, Wl0, bl0.reshape(1, _D), Wr0)
    h1 = _tc_norm(h0, st0, g0.reshape(1, _D), be0.reshape(1, _D))

    acc1 = _sc_aggregate(h1, eidx, False)
    h2, st1 = _tc_combine(acc1, cnt, h1, Wl1, bl1.reshape(1, _D), Wr1)

    return _tc_final(h2, st1, g1.reshape(1, _D), be1.reshape(1, _D),
                     batch.reshape(_N, 1), fcW1, fcb1.reshape(1, _D // 2),
                     fcW2, fcb2.reshape(1, 1))


# speedup vs baseline: 1.7400x; 1.0235x over previous
import functools

import jax
import jax.numpy as jnp
from jax import lax
from jax.experimental import pallas as pl
from jax.experimental.pallas import tpu as pltpu
from jax.experimental.pallas import tpu_sc as plsc

_N = 10000
_E = 320000
_D = 128
_G = 128
_EPS = 1e-5

_NC = 2
_NS = 16
_NW = _NC * _NS
_C = 80
_NCH = 125
_EP = _NW * _NCH * _C
_NSH = _N
_RPS = 624
_RTAIL = _N - _NS * _RPS
_CNT_CH = 1000
_R = 400


def _zero_acc(z2_hbm, acc_sh, s):
    zoff = pl.multiple_of(s * _RPS, 8)
    pltpu.sync_copy(z2_hbm.at[pl.ds(zoff, _RPS)],
                    acc_sh.at[pl.ds(zoff, _RPS)])

    @pl.when(s == 0)
    def _():
        pltpu.sync_copy(z2_hbm.at[pl.ds(_NS * _RPS, _RTAIL)],
                        acc_sh.at[pl.ds(_NS * _RPS, _RTAIL)])


def _acc_out_copy(acc_sh, acc_out, c, s):
    zoff = pl.multiple_of(s * _RPS, 8)
    ooff = pl.multiple_of(c * _N + s * _RPS, 8)
    pltpu.sync_copy(acc_sh.at[pl.ds(zoff, _RPS)],
                    acc_out.at[pl.ds(ooff, _RPS)])

    @pl.when(s == 0)
    def _():
        toff = pl.multiple_of(c * _N + _NS * _RPS, 8)
        pltpu.sync_copy(acc_sh.at[pl.ds(_NS * _RPS, _RTAIL)],
                        acc_out.at[pl.ds(toff, _RTAIL)])


def _make_agg_body(with_cnt):
    def body(x_hbm, eidx_hbm, z2_hbm, *refs):
        if with_cnt:
            (z1_hbm, acc_out, cnt_out, idx_a, idx_b, rows_a, rows_b, ones_v,
             cntb_v, acc_sh, cnt_sh, isem_a, isem_b, gsem_a, gsem_b) = refs
        else:
            (acc_out, idx_a, idx_b, rows_a, rows_b, acc_sh,
             isem_a, isem_b, gsem_a, gsem_b) = refs
        c = lax.axis_index("c")
        s = lax.axis_index("s")
        wid = c * _NS + s
        base = wid * _NCH

        pltpu.async_copy(eidx_hbm.at[base], idx_a, isem_a)
        pltpu.async_copy(eidx_hbm.at[base + 1], idx_b, isem_b)

        _zero_acc(z2_hbm, acc_sh, s)

        if with_cnt:
            @pl.when(s < _N // _CNT_CH)
            def _():
                coff = pl.multiple_of(s * _CNT_CH, 8)
                pltpu.sync_copy(z1_hbm.at[pl.ds(coff, _CNT_CH)], cntb_v)
                pltpu.sync_copy(cntb_v, cnt_sh.at[pl.ds(coff, _CNT_CH)])

            ones16 = jnp.full((16,), 1.0, jnp.float32)
            for j in range(_C // 16):
                ones_v[pl.ds(j * 16, 16)] = ones16

        pltpu.make_async_copy(eidx_hbm.at[base], idx_a, isem_a).wait()
        pltpu.async_copy(x_hbm.at[idx_a.at[0]], rows_a, gsem_a)
        plsc.subcore_barrier()

        def scatter(rows, idx):
            pltpu.sync_copy(rows, acc_sh.at[idx.at[1]], add=True)
            if with_cnt:
                pltpu.sync_copy(ones_v, cnt_sh.at[idx.at[1]], add=True)

        def pair(k, carry):
            i0 = 2 * k
            pltpu.make_async_copy(eidx_hbm.at[base + i0 + 1], idx_b,
                                  isem_b).wait()
            pltpu.async_copy(x_hbm.at[idx_b.at[0]], rows_b, gsem_b)
            pltpu.make_async_copy(x_hbm.at[idx_a.at[0]], rows_a,
                                  gsem_a).wait()
            scatter(rows_a, idx_a)
            pltpu.async_copy(eidx_hbm.at[base + i0 + 2], idx_a, isem_a)
            pltpu.make_async_copy(eidx_hbm.at[base + i0 + 2], idx_a,
                                  isem_a).wait()
            pltpu.async_copy(x_hbm.at[idx_a.at[0]], rows_a, gsem_a)
            pltpu.make_async_copy(x_hbm.at[idx_b.at[0]], rows_b,
                                  gsem_b).wait()
            scatter(rows_b, idx_b)

            @pl.when(k < _NCH // 2 - 1)
            def _():
                pltpu.async_copy(eidx_hbm.at[base + i0 + 3], idx_b, isem_b)

            return carry

        lax.fori_loop(0, _NCH // 2, pair, 0)

        pltpu.make_async_copy(x_hbm.at[idx_a.at[0]], rows_a, gsem_a).wait()
        scatter(rows_a, idx_a)

        plsc.subcore_barrier()

        _acc_out_copy(acc_sh, acc_out, c, s)

        if with_cnt:
            @pl.when(s < _N // _CNT_CH)
            def _():
                coff = pl.multiple_of(s * _CNT_CH, 8)
                off = pl.multiple_of(c * _N + s * _CNT_CH, 8)
                pltpu.sync_copy(cnt_sh.at[pl.ds(coff, _CNT_CH)], cntb_v)
                pltpu.sync_copy(cntb_v, cnt_out.at[pl.ds(off, _CNT_CH)])

    return body


@functools.lru_cache(maxsize=None)
def _build_agg(with_cnt):
    mesh = plsc.VectorSubcoreMesh(core_axis_name="c", subcore_axis_name="s",
                                  num_cores=_NC, num_subcores=_NS)
    if with_cnt:
        return pl.kernel(
            _make_agg_body(True),
            out_type=(jax.ShapeDtypeStruct((_NC * _N, _D), jnp.float32),
                      jax.ShapeDtypeStruct((_NC * _N,), jnp.float32)),
            mesh=mesh,
            scratch_types=[
                pltpu.VMEM((2, _C), jnp.int32),
                pltpu.VMEM((2, _C), jnp.int32),
                pltpu.VMEM((_C, _D), jnp.float32),
                pltpu.VMEM((_C, _D), jnp.float32),
                pltpu.VMEM((_C,), jnp.float32),
                pltpu.VMEM((_CNT_CH,), jnp.float32),
                pltpu.VMEM_SHARED((_NSH, _D), jnp.float32),
                pltpu.VMEM_SHARED((_NSH,), jnp.float32),
                pltpu.SemaphoreType.DMA,
                pltpu.SemaphoreType.DMA,
                pltpu.SemaphoreType.DMA,
                pltpu.SemaphoreType.DMA,
            ],
        )
    return pl.kernel(
        _make_agg_body(False),
        out_type=jax.ShapeDtypeStruct((_NC * _N, _D), jnp.float32),
        mesh=mesh,
        scratch_types=[
            pltpu.VMEM((2, _C), jnp.int32),
            pltpu.VMEM((2, _C), jnp.int32),
            pltpu.VMEM((_C, _D), jnp.float32),
            pltpu.VMEM((_C, _D), jnp.float32),
            pltpu.VMEM_SHARED((_NSH, _D), jnp.float32),
            pltpu.SemaphoreType.DMA,
            pltpu.SemaphoreType.DMA,
            pltpu.SemaphoreType.DMA,
            pltpu.SemaphoreType.DMA,
        ],
    )


def _pack_edges(edge_index):
    return jnp.stack([edge_index[0].reshape(_NW * _NCH, _C),
                      edge_index[1].reshape(_NW * _NCH, _C)], axis=1)


def _sc_aggregate(x, eidx, with_cnt):
    z2 = jnp.zeros((_N, _D), jnp.float32)
    if with_cnt:
        z1 = jnp.zeros((_N,), jnp.float32)
        acc, cnt = _build_agg(True)(x, eidx, z2, z1)
        return acc.reshape(_NC, _N, _D), cnt.reshape(_NC, _N, 1)
    acc = _build_agg(False)(x, eidx, z2)
    return acc.reshape(_NC, _N, _D)


def _combine_pass(acc0_r, acc1_r, cnt0_r, cnt1_r, x_r, wl_r, bl_r, wr_r,
                  h_sc, st_sc, i):
    agg = acc0_r[0] + acc1_r[0]
    cnt = cnt0_r[0] + cnt1_r[0]
    mean = agg / jnp.maximum(cnt, 1.0)
    h = (jnp.dot(mean, wl_r[...], preferred_element_type=jnp.float32)
         + jnp.dot(x_r[...], wr_r[...], preferred_element_type=jnp.float32)
         + bl_r[...])
    h_sc[pl.ds(i * _R, _R), :] = h
    ssum = jnp.sum(h, axis=0, keepdims=True)
    ssq = jnp.sum(h * h, axis=0, keepdims=True)
    st = jnp.concatenate([ssum, ssq, jnp.zeros((6, _D), jnp.float32)], axis=0)

    @pl.when(i == 0)
    def _():
        st_sc[...] = st

    @pl.when(i > 0)
    def _():
        st_sc[...] = st_sc[...] + st


def _normed(h_sc, st_sc, g_r, be_r, i):
    st = st_sc[...]
    mu = st[0:1, :] * (1.0 / _N)
    var = st[1:2, :] * (1.0 / _N) - mu * mu
    h = h_sc[pl.ds(i * _R, _R), :]
    return jnp.maximum(
        g_r[...] * (h - mu) * lax.rsqrt(var + _EPS) + be_r[...], 0.0)


def _layer1_body(acc0_r, acc1_r, cnt0_r, cnt1_r, x_r, wl_r, bl_r, wr_r,
                 g_r, be_r, o_r, h_sc, st_sc):
    p = pl.program_id(0)
    i = pl.program_id(1)

    @pl.when(p == 0)
    def _():
        _combine_pass(acc0_r, acc1_r, cnt0_r, cnt1_r, x_r, wl_r, bl_r, wr_r,
                      h_sc, st_sc, i)

    @pl.when(p == 1)
    def _():
        o_r[...] = _normed(h_sc, st_sc, g_r, be_r, i)


def _tc_layer1(acc, cnt, x, wl, bl, wr, g, be):
    return pl.pallas_call(
        _layer1_body,
        grid=(2, _N // _R),
        in_specs=[
            pl.BlockSpec((1, _R, _D), lambda p, i: (0, i * (1 - p), 0)),
            pl.BlockSpec((1, _R, _D), lambda p, i: (1, i * (1 - p), 0)),
            pl.BlockSpec((1, _R, 1), lambda p, i: (0, i * (1 - p), 0)),
            pl.BlockSpec((1, _R, 1), lambda p, i: (1, i * (1 - p), 0)),
            pl.BlockSpec((_R, _D), lambda p, i: (i * (1 - p), 0)),
            pl.BlockSpec((_D, _D), lambda p, i: (0, 0)),
            pl.BlockSpec((1, _D), lambda p, i: (0, 0)),
            pl.BlockSpec((_D, _D), lambda p, i: (0, 0)),
            pl.BlockSpec((1, _D), lambda p, i: (0, 0)),
            pl.BlockSpec((1, _D), lambda p, i: (0, 0)),
        ],
        out_specs=pl.BlockSpec((_R, _D), lambda p, i: (i * p, 0)),
        out_shape=jax.ShapeDtypeStruct((_N, _D), jnp.float32),
        scratch_shapes=[pltpu.VMEM((_N, _D), jnp.float32),
                        pltpu.VMEM((8, _D), jnp.float32)],
    )(acc, acc, cnt, cnt, x, wl, bl, wr, g, be)


def _layer2_body(acc0_r, acc1_r, cnt0_r, cnt1_r, x_r, wl_r, bl_r, wr_r,
                 g_r, be_r, b_r, w1_r, b1_r, w2_r, b2_r, o_r,
                 h_sc, st_sc, pool_sc):
    p = pl.program_id(0)
    i = pl.program_id(1)

    @pl.when(p == 0)
    def _():
        _combine_pass(acc0_r, acc1_r, cnt0_r, cnt1_r, x_r, wl_r, bl_r, wr_r,
                      h_sc, st_sc, i)

    @pl.when(p == 1)
    def _():
        h2 = _normed(h_sc, st_sc, g_r, be_r, i)
        onehot = (b_r[...] == lax.broadcasted_iota(jnp.int32, (_R, _G), 1)
                  ).astype(jnp.float32)
        pp = lax.dot_general(onehot, h2, (((0,), (0,)), ((), ())),
                             preferred_element_type=jnp.float32)

        @pl.when(i == 0)
        def _():
            pool_sc[...] = pp

        @pl.when(i > 0)
        def _():
            pool_sc[...] = pool_sc[...] + pp

        @pl.when(i == pl.num_programs(1) - 1)
        def _():
            t = jnp.maximum(
                jnp.dot(pool_sc[...], w1_r[...],
                        preferred_element_type=jnp.float32) + b1_r[...], 0.0)
            o_r[...] = (jnp.dot(t, w2_r[...],
                                preferred_element_type=jnp.float32) + b2_r[...])


def _tc_layer2(acc, cnt, h1, wl, bl, wr, g, be, batch2d, w1, b1, w2, b2):
    return pl.pallas_call(
        _layer2_body,
        grid=(2, _N // _R),
        in_specs=[
            pl.BlockSpec((1, _R, _D), lambda p, i: (0, i * (1 - p), 0)),
            pl.BlockSpec((1, _R, _D), lambda p, i: (1, i * (1 - p), 0)),
            pl.BlockSpec((1, _R, 1), lambda p, i: (0, i * (1 - p), 0)),
            pl.BlockSpec((1, _R, 1), lambda p, i: (1, i * (1 - p), 0)),
            pl.BlockSpec((_R, _D), lambda p, i: (i * (1 - p), 0)),
            pl.BlockSpec((_D, _D), lambda p, i: (0, 0)),
            pl.BlockSpec((1, _D), lambda p, i: (0, 0)),
            pl.BlockSpec((_D, _D), lambda p, i: (0, 0)),
            pl.BlockSpec((1, _D), lambda p, i: (0, 0)),
            pl.BlockSpec((1, _D), lambda p, i: (0, 0)),
            pl.BlockSpec((_R, 1), lambda p, i: (i * p, 0)),
            pl.BlockSpec((_D, _D // 2), lambda p, i: (0, 0)),
            pl.BlockSpec((1, _D // 2), lambda p, i: (0, 0)),
            pl.BlockSpec((_D // 2, 1), lambda p, i: (0, 0)),
            pl.BlockSpec((1, 1), lambda p, i: (0, 0)),
        ],
        out_specs=pl.BlockSpec((_G, 1), lambda p, i: (0, 0)),
        out_shape=jax.ShapeDtypeStruct((_G, 1), jnp.float32),
        scratch_shapes=[pltpu.VMEM((_N, _D), jnp.float32),
                        pltpu.VMEM((8, _D), jnp.float32),
                        pltpu.VMEM((_G, _G), jnp.float32)],
    )(acc, acc, cnt, cnt, h1, wl, bl, wr, g, be, batch2d, w1, b1, w2, b2)


def kernel(x, edge_index, batch, Wl0, bl0, Wr0, g0, be0,
           Wl1, bl1, Wr1, g1, be1, fcW1, fcb1, fcW2, fcb2):
    eidx = _pack_edges(edge_index)

    acc0, cnt = _sc_aggregate(x, eidx, True)
    h1 = _tc_layer1(acc0, cnt, x, Wl0, bl0.reshape(1, _D), Wr0,
                    g0.reshape(1, _D), be0.reshape(1, _D))

    acc1 = _sc_aggregate(h1, eidx, False)
    return _tc_layer2(acc1, cnt, h1, Wl1, bl1.reshape(1, _D), Wr1,
                      g1.reshape(1, _D), be1.reshape(1, _D),
                      batch.reshape(_N, 1), fcW1, fcb1.reshape(1, _D // 2),
                      fcW2, fcb2.reshape(1, 1))
